# baseline (speedup 1.0000x reference)
import functools

import jax
import jax.numpy as jnp
from jax.experimental import pallas as pl
from jax.experimental.pallas import tpu as pltpu
from jax.experimental.pallas import tpu_sc as plsc

N_NODES = 10000
N_EDGES = 320000
D_IN = 128
D_HID = 256
D_OUT = 64

NSUB = 16
CHUNK = 40
IDX_BLK = 50
N_CHUNKS = N_EDGES // CHUNK
N_BLKS = N_CHUNKS // IDX_BLK
N_PAD = 10240
ROWS_PER_SUB = N_PAD // NSUB
ZROWS = 32


def _fill(ref, nrows, ncols, value):
    @pl.loop(0, nrows)
    def _(r):
        @pl.loop(0, ncols, step=16)
        def _(cc):
            ref[r, pl.ds(cc, 16)] = jnp.full((16,), value, jnp.float32)


def _zero_share(zbuf, acc, sid):
    for b in range(ROWS_PER_SUB // ZROWS):
        pltpu.sync_copy(zbuf, acc.at[pl.ds(sid * ROWS_PER_SUB + b * ZROWS,
                                           ZROWS)])


GRP = 2
NGRP = IDX_BLK // GRP


def _edge_pipeline(x_hbm, srcv, dstv, bufs_a, bufs_b, acc, gsem, ssem):
    def fire_g(bufs, g):
        for i in range(GRP):
            pltpu.async_copy(x_hbm.at[srcv.at[GRP * g + i]], bufs[i], gsem)

    def drain_g(bufs):
        for i in range(GRP):
            pltpu.make_async_copy(x_hbm.at[srcv.at[0]], bufs[i], gsem).wait()

    def fire_s(bufs, g):
        for i in range(GRP):
            pltpu.async_copy(bufs[i], acc.at[dstv.at[GRP * g + i]], ssem,
                             add=True)

    def drain_s(bufs):
        for i in range(GRP):
            pltpu.make_async_copy(bufs[i], acc.at[dstv.at[0]], ssem).wait()

    fire_g(bufs_a, 0)

    @pl.loop(0, (NGRP - 1) // 2)
    def _(q):
        g0 = 2 * q
        drain_g(bufs_a)

        @pl.when(g0 > 0)
        def _():
            drain_s(bufs_b)

        fire_g(bufs_b, g0 + 1)
        fire_s(bufs_a, g0)
        drain_g(bufs_b)
        drain_s(bufs_a)
        fire_g(bufs_a, g0 + 2)
        fire_s(bufs_b, g0 + 1)

    drain_g(bufs_a)
    drain_s(bufs_b)
    fire_s(bufs_a, NGRP - 1)
    drain_s(bufs_a)


def _make_segsum1():
    mesh = plsc.VectorSubcoreMesh(core_axis_name="c", subcore_axis_name="s")
    blk_per_core = N_BLKS // 2
    n_outer = blk_per_core // NSUB

    out_type = (jax.ShapeDtypeStruct((N_PAD, D_IN), jnp.float32),
                jax.ShapeDtypeStruct((N_PAD, D_IN), jnp.float32))
    scratch = [
        pltpu.VMEM((IDX_BLK, CHUNK), jnp.int32),
        pltpu.VMEM((IDX_BLK, CHUNK), jnp.int32),
        pltpu.VMEM((CHUNK, D_IN), jnp.float32),
        pltpu.VMEM((CHUNK, D_IN), jnp.float32),
        pltpu.VMEM((CHUNK, D_IN), jnp.float32),
        pltpu.VMEM((CHUNK, D_IN), jnp.float32),
        pltpu.VMEM((ZROWS, D_IN), jnp.float32),
        pltpu.VMEM_SHARED((N_PAD, D_IN), jnp.float32),
        pltpu.SemaphoreType.DMA,
        pltpu.SemaphoreType.DMA,
    ]

    @functools.partial(pl.kernel, mesh=mesh, out_type=out_type,
                       scratch_types=scratch)
    def seg_kernel(src_hbm, dst_hbm, x_hbm, oa, ob,
                   srcv, dstv, ra0, ra1, rb0, rb1, zbuf, acc, gsem, ssem):
        cid = jax.lax.axis_index("c")
        sid = jax.lax.axis_index("s")

        _fill(zbuf, ZROWS, D_IN, 0.0)
        _zero_share(zbuf, acc, sid)
        plsc.subcore_barrier()

        @pl.loop(0, n_outer)
        def _(t):
            blk = cid * blk_per_core + sid * n_outer + t
            pltpu.sync_copy(src_hbm.at[blk], srcv)
            pltpu.sync_copy(dst_hbm.at[blk], dstv)
            _edge_pipeline(x_hbm, srcv, dstv, (ra0, ra1), (rb0, rb1),
                           acc, gsem, ssem)

        plsc.subcore_barrier()

        rs = pl.ds(sid * ROWS_PER_SUB, ROWS_PER_SUB)

        @pl.when(cid == 0)
        def _():
            pltpu.sync_copy(acc.at[rs], oa.at[rs])

        @pl.when(cid == 1)
        def _():
            pltpu.sync_copy(acc.at[rs], ob.at[rs])

    return seg_kernel


def _make_segdeg():
    mesh = plsc.VectorSubcoreMesh(core_axis_name="c", subcore_axis_name="s")
    blk_per_core = N_BLKS // 2
    n_outer = blk_per_core // NSUB

    out_type = (jax.ShapeDtypeStruct((N_PAD, 128), jnp.float32),
                jax.ShapeDtypeStruct((N_PAD, 128), jnp.float32))
    scratch = [
        pltpu.VMEM((IDX_BLK, CHUNK), jnp.int32),
        pltpu.VMEM((CHUNK, 128), jnp.float32),
        pltpu.VMEM((ZROWS, 128), jnp.float32),
        pltpu.VMEM_SHARED((N_PAD, 128), jnp.float32),
        pltpu.SemaphoreType.DMA,
    ]

    @functools.partial(pl.kernel, mesh=mesh, out_type=out_type,
                       scratch_types=scratch)
    def deg_kernel(dst_hbm, da, db, dstv, ones, zbuf, dacc, sem):
        cid = jax.lax.axis_index("c")
        sid = jax.lax.axis_index("s")

        _fill(zbuf, ZROWS, 128, 0.0)
        _fill(ones, CHUNK, 128, 1.0)
        _zero_share(zbuf, dacc, sid)
        plsc.subcore_barrier()

        @pl.loop(0, n_outer)
        def _(t):
            blk = cid * blk_per_core + sid * n_outer + t
            pltpu.sync_copy(dst_hbm.at[blk], dstv)

            @pl.loop(0, IDX_BLK)
            def _(j):
                pltpu.async_copy(ones, dacc.at[dstv.at[j]], sem, add=True)

            @pl.loop(0, IDX_BLK)
            def _(j):
                pltpu.make_async_copy(ones, dacc.at[dstv.at[0]], sem).wait()

        plsc.subcore_barrier()

        rs = pl.ds(sid * ROWS_PER_SUB, ROWS_PER_SUB)

        @pl.when(cid == 0)
        def _():
            pltpu.sync_copy(dacc.at[rs], da.at[rs])

        @pl.when(cid == 1)
        def _():
            pltpu.sync_copy(dacc.at[rs], db.at[rs])

    return deg_kernel


def _make_segsum2():
    mesh = plsc.VectorSubcoreMesh(core_axis_name="c", subcore_axis_name="s")
    dh = D_HID // 2
    n_outer = N_BLKS // NSUB

    out_type = (jax.ShapeDtypeStruct((N_PAD, dh), jnp.float32),
                jax.ShapeDtypeStruct((N_PAD, dh), jnp.float32))
    scratch = [
        pltpu.VMEM((IDX_BLK, CHUNK), jnp.int32),
        pltpu.VMEM((IDX_BLK, CHUNK), jnp.int32),
        pltpu.VMEM((CHUNK, dh), jnp.float32),
        pltpu.VMEM((CHUNK, dh), jnp.float32),
        pltpu.VMEM((CHUNK, dh), jnp.float32),
        pltpu.VMEM((CHUNK, dh), jnp.float32),
        pltpu.VMEM((ZROWS, dh), jnp.float32),
        pltpu.VMEM_SHARED((N_PAD, dh), jnp.float32),
        pltpu.SemaphoreType.DMA,
        pltpu.SemaphoreType.DMA,
    ]

    @functools.partial(pl.kernel, mesh=mesh, out_type=out_type,
                       scratch_types=scratch)
    def seg_kernel(src_hbm, dst_hbm, hlo_hbm, hhi_hbm, olo, ohi,
                   srcv, dstv, ra0, ra1, rb0, rb1, zbuf, acc, gsem, ssem):
        cid = jax.lax.axis_index("c")
        sid = jax.lax.axis_index("s")

        _fill(zbuf, ZROWS, dh, 0.0)
        _zero_share(zbuf, acc, sid)
        plsc.subcore_barrier()

        @pl.loop(0, n_outer)
        def _(t):
            blk = sid * n_outer + t
            pltpu.sync_copy(src_hbm.at[blk], srcv)
            pltpu.sync_copy(dst_hbm.at[blk], dstv)

            @pl.when(cid == 0)
            def _():
                _edge_pipeline(hlo_hbm, srcv, dstv, (ra0, ra1), (rb0, rb1),
                               acc, gsem, ssem)

            @pl.when(cid == 1)
            def _():
                _edge_pipeline(hhi_hbm, srcv, dstv, (ra0, ra1), (rb0, rb1),
                               acc, gsem, ssem)

        plsc.subcore_barrier()

        rs = pl.ds(sid * ROWS_PER_SUB, ROWS_PER_SUB)

        @pl.when(cid == 0)
        def _():
            pltpu.sync_copy(acc.at[rs], olo.at[rs])

        @pl.when(cid == 1)
        def _():
            pltpu.sync_copy(acc.at[rs], ohi.at[rs])

    return seg_kernel


_segsum1 = _make_segsum1()
_segsum2 = _make_segsum2()
_segdeg = _make_segdeg()

_TC_BLK = 1000


def _sage1(agg_a, agg_b, dega, degb, x, wl, wr, bl):
    def body(aa, ab, da, db, xr, wll, wrr, b, out):
        deg = jnp.maximum(da[...][:, :1] + db[...][:, :1], 1.0)
        h = jnp.dot((aa[...] + ab[...]) / deg, wll[...],
                    preferred_element_type=jnp.float32)
        h = h + jnp.dot(xr[...], wrr[...], preferred_element_type=jnp.float32)
        h = jnp.maximum(h + b[...], 0.0)
        out[0] = h[:, :D_HID // 2]
        out[1] = h[:, D_HID // 2:]

    return pl.pallas_call(
        body,
        grid=(N_NODES // _TC_BLK,),
        in_specs=[
            pl.BlockSpec((_TC_BLK, D_IN), lambda i: (i, 0)),
            pl.BlockSpec((_TC_BLK, D_IN), lambda i: (i, 0)),
            pl.BlockSpec((_TC_BLK, 128), lambda i: (i, 0)),
            pl.BlockSpec((_TC_BLK, 128), lambda i: (i, 0)),
            pl.BlockSpec((_TC_BLK, D_IN), lambda i: (i, 0)),
            pl.BlockSpec((D_IN, D_HID), lambda i: (0, 0)),
            pl.BlockSpec((D_IN, D_HID), lambda i: (0, 0)),
            pl.BlockSpec((1, D_HID), lambda i: (0, 0)),
        ],
        out_specs=pl.BlockSpec((2, _TC_BLK, D_HID // 2), lambda i: (0, i, 0)),
        out_shape=jax.ShapeDtypeStruct((2, N_NODES, D_HID // 2), jnp.float32),
    )(agg_a, agg_b, dega, degb, x, wl, wr, bl)


def _sage2_final(a2lo, a2hi, dega, degb, h1lo, h1hi,
                 wla, wlb, wra, wrb, bl, wf, bf):
    def body(alo, ahi, da, db, hlo, hhi, wa, wb, wr1, wr2, b, wff, bff, out):
        deg = jnp.maximum(da[...][:, :1] + db[...][:, :1], 1.0)
        h = jnp.dot(alo[...] / deg, wa[...], preferred_element_type=jnp.float32)
        h = h + jnp.dot(ahi[...] / deg, wb[...],
                        preferred_element_type=jnp.float32)
        h = h + jnp.dot(hlo[...], wr1[...], preferred_element_type=jnp.float32)
        h = h + jnp.dot(hhi[...], wr2[...], preferred_element_type=jnp.float32)
        h = jnp.maximum(h + b[...], 0.0)
        o = jnp.dot(h, wff[...], preferred_element_type=jnp.float32) + bff[...]
        s = o - jnp.max(o, axis=1, keepdims=True)
        out[...] = s - jnp.log(jnp.sum(jnp.exp(s), axis=1, keepdims=True))

    half = D_HID // 2
    return pl.pallas_call(
        body,
        grid=(N_NODES // _TC_BLK,),
        in_specs=[
            pl.BlockSpec((_TC_BLK, half), lambda i: (i, 0)),
            pl.BlockSpec((_TC_BLK, half), lambda i: (i, 0)),
            pl.BlockSpec((_TC_BLK, 128), lambda i: (i, 0)),
            pl.BlockSpec((_TC_BLK, 128), lambda i: (i, 0)),
            pl.BlockSpec((_TC_BLK, half), lambda i: (i, 0)),
            pl.BlockSpec((_TC_BLK, half), lambda i: (i, 0)),
            pl.BlockSpec((half, D_HID), lambda i: (0, 0)),
            pl.BlockSpec((half, D_HID), lambda i: (0, 0)),
            pl.BlockSpec((half, D_HID), lambda i: (0, 0)),
            pl.BlockSpec((half, D_HID), lambda i: (0, 0)),
            pl.BlockSpec((1, D_HID), lambda i: (0, 0)),
            pl.BlockSpec((D_HID, D_OUT), lambda i: (0, 0)),
            pl.BlockSpec((1, D_OUT), lambda i: (0, 0)),
        ],
        out_specs=pl.BlockSpec((_TC_BLK, D_OUT), lambda i: (i, 0)),
        out_shape=jax.ShapeDtypeStruct((N_NODES, D_OUT), jnp.float32),
    )(a2lo, a2hi, dega, degb, h1lo, h1hi, wla, wlb, wra, wrb, bl, wf, bf)


def kernel(x, edge_index, Wl1, bl1, Wr1, Wl2, bl2, Wr2, Wf, bf):
    src3 = edge_index[0].astype(jnp.int32).reshape(N_BLKS, IDX_BLK, CHUNK)
    dst3 = edge_index[1].astype(jnp.int32).reshape(N_BLKS, IDX_BLK, CHUNK)

    dega, degb = _segdeg(dst3)
    a1a, a1b = _segsum1(src3, dst3, x)
    h13 = _sage1(a1a, a1b, dega, degb, x, Wl1.T, Wr1.T, bl1.reshape(1, -1))
    a2lo, a2hi = _segsum2(src3, dst3, h13[0], h13[1])
    out = _sage2_final(a2lo, a2hi, dega, degb, h13[0], h13[1],
                       Wl2.T[:D_HID // 2], Wl2.T[D_HID // 2:],
                       Wr2.T[:D_HID // 2], Wr2.T[D_HID // 2:],
                       bl2.reshape(1, -1), Wf.T, bf.reshape(1, -1))
    return out


# speedup vs baseline: 1.0224x; 1.0224x over previous
import functools

import jax
import jax.numpy as jnp
from jax.experimental import pallas as pl
from jax.experimental.pallas import tpu as pltpu
from jax.experimental.pallas import tpu_sc as plsc

N_NODES = 10000
N_EDGES = 320000
D_IN = 128
D_HID = 256
D_OUT = 64

NSUB = 16
CHUNK = 80
IDX_BLK = 25
N_CHUNKS = N_EDGES // CHUNK
N_BLKS = N_CHUNKS // IDX_BLK
N_PAD = 10240
ROWS_PER_SUB = N_PAD // NSUB
ZROWS = 32


def _fill(ref, nrows, ncols, value):
    @pl.loop(0, nrows)
    def _(r):
        @pl.loop(0, ncols, step=16)
        def _(cc):
            ref[r, pl.ds(cc, 16)] = jnp.full((16,), value, jnp.float32)


def _zero_share(zbuf, acc, sid):
    for b in range(ROWS_PER_SUB // ZROWS):
        pltpu.sync_copy(zbuf, acc.at[pl.ds(sid * ROWS_PER_SUB + b * ZROWS,
                                           ZROWS)])


def _edge_pipeline(x_hbm, srcv, dstv, rows_a, rows_b, acc, gsem, ssem):
    pltpu.async_copy(x_hbm.at[srcv.at[0]], rows_a, gsem)

    @pl.loop(0, IDX_BLK // 2)
    def _(p):
        j0 = 2 * p
        pltpu.make_async_copy(x_hbm.at[srcv.at[0]], rows_a, gsem).wait()

        @pl.when(p > 0)
        def _():
            pltpu.make_async_copy(rows_b, acc.at[dstv.at[0]], ssem).wait()

        hg_b = pltpu.async_copy(x_hbm.at[srcv.at[j0 + 1]], rows_b, gsem)
        hs_a = pltpu.async_copy(rows_a, acc.at[dstv.at[j0]], ssem, add=True)
        hg_b.wait()
        hs_a.wait()
        pltpu.async_copy(x_hbm.at[srcv.at[j0 + 2]], rows_a, gsem)
        pltpu.async_copy(rows_b, acc.at[dstv.at[j0 + 1]], ssem, add=True)

    pltpu.make_async_copy(x_hbm.at[srcv.at[0]], rows_a, gsem).wait()
    pltpu.make_async_copy(rows_b, acc.at[dstv.at[0]], ssem).wait()
    pltpu.sync_copy(rows_a, acc.at[dstv.at[IDX_BLK - 1]], add=True)


def _make_segsum1():
    mesh = plsc.VectorSubcoreMesh(core_axis_name="c", subcore_axis_name="s")
    blk_per_core = N_BLKS // 2
    n_outer = blk_per_core // NSUB

    out_type = (jax.ShapeDtypeStruct((N_PAD, D_IN), jnp.float32),
                jax.ShapeDtypeStruct((N_PAD, D_IN), jnp.float32),
                jax.ShapeDtypeStruct((N_PAD, D_IN), jnp.float32),
                jax.ShapeDtypeStruct((N_PAD, D_IN), jnp.float32))
    scratch = [
        pltpu.VMEM((IDX_BLK, CHUNK), jnp.int32),
        pltpu.VMEM((IDX_BLK, CHUNK), jnp.int32),
        pltpu.VMEM((CHUNK, D_IN), jnp.float32),
        pltpu.VMEM((CHUNK, D_IN), jnp.float32),
        pltpu.VMEM((ZROWS, D_IN), jnp.float32),
        pltpu.VMEM_SHARED((N_PAD, D_IN), jnp.float32),
        pltpu.SemaphoreType.DMA,
        pltpu.SemaphoreType.DMA,
    ]

    @functools.partial(pl.kernel, mesh=mesh, out_type=out_type,
                       scratch_types=scratch)
    def seg_kernel(src_hbm, dst_hbm, x_hbm, oa, ob, da, db,
                   srcv, dstv, rows_a, rows_b, zbuf, acc, gsem, ssem):
        cid = jax.lax.axis_index("c")
        sid = jax.lax.axis_index("s")

        _fill(zbuf, ZROWS, D_IN, 0.0)
        _fill(rows_a, CHUNK, D_IN, 1.0)
        _zero_share(zbuf, acc, sid)
        plsc.subcore_barrier()

        rs = pl.ds(sid * ROWS_PER_SUB, ROWS_PER_SUB)

        @pl.loop(0, n_outer)
        def _(t):
            blk = cid * blk_per_core + sid * n_outer + t
            pltpu.sync_copy(dst_hbm.at[blk], dstv)

            @pl.loop(0, IDX_BLK)
            def _(j):
                pltpu.async_copy(rows_a, acc.at[dstv.at[j]], ssem, add=True)

            @pl.loop(0, IDX_BLK)
            def _(j):
                pltpu.make_async_copy(rows_a, acc.at[dstv.at[0]],
                                      ssem).wait()

        plsc.subcore_barrier()

        @pl.when(cid == 0)
        def _():
            pltpu.sync_copy(acc.at[rs], da.at[rs])

        @pl.when(cid == 1)
        def _():
            pltpu.sync_copy(acc.at[rs], db.at[rs])

        plsc.subcore_barrier()
        _zero_share(zbuf, acc, sid)
        plsc.subcore_barrier()

        @pl.loop(0, n_outer)
        def _(t):
            blk = cid * blk_per_core + sid * n_outer + t
            pltpu.sync_copy(src_hbm.at[blk], srcv)
            pltpu.sync_copy(dst_hbm.at[blk], dstv)
            _edge_pipeline(x_hbm, srcv, dstv, rows_a, rows_b, acc,
                           gsem, ssem)

        plsc.subcore_barrier()

        @pl.when(cid == 0)
        def _():
            pltpu.sync_copy(acc.at[rs], oa.at[rs])

        @pl.when(cid == 1)
        def _():
            pltpu.sync_copy(acc.at[rs], ob.at[rs])

    return seg_kernel


def _make_segsum2():
    mesh = plsc.VectorSubcoreMesh(core_axis_name="c", subcore_axis_name="s")
    dh = D_HID // 2
    n_outer = N_BLKS // NSUB

    out_type = (jax.ShapeDtypeStruct((N_PAD, dh), jnp.float32),
                jax.ShapeDtypeStruct((N_PAD, dh), jnp.float32))
    scratch = [
        pltpu.VMEM((IDX_BLK, CHUNK), jnp.int32),
        pltpu.VMEM((IDX_BLK, CHUNK), jnp.int32),
        pltpu.VMEM((CHUNK, dh), jnp.float32),
        pltpu.VMEM((CHUNK, dh), jnp.float32),
        pltpu.VMEM((ZROWS, dh), jnp.float32),
        pltpu.VMEM_SHARED((N_PAD, dh), jnp.float32),
        pltpu.SemaphoreType.DMA,
        pltpu.SemaphoreType.DMA,
    ]

    @functools.partial(pl.kernel, mesh=mesh, out_type=out_type,
                       scratch_types=scratch)
    def seg_kernel(src_hbm, dst_hbm, hlo_hbm, hhi_hbm, olo, ohi,
                   srcv, dstv, rows_a, rows_b, zbuf, acc, gsem, ssem):
        cid = jax.lax.axis_index("c")
        sid = jax.lax.axis_index("s")

        _fill(zbuf, ZROWS, dh, 0.0)
        _zero_share(zbuf, acc, sid)
        plsc.subcore_barrier()

        @pl.loop(0, n_outer)
        def _(t):
            blk = sid * n_outer + t
            pltpu.sync_copy(src_hbm.at[blk], srcv)
            pltpu.sync_copy(dst_hbm.at[blk], dstv)

            @pl.when(cid == 0)
            def _():
                _edge_pipeline(hlo_hbm, srcv, dstv, rows_a, rows_b, acc,
                               gsem, ssem)

            @pl.when(cid == 1)
            def _():
                _edge_pipeline(hhi_hbm, srcv, dstv, rows_a, rows_b, acc,
                               gsem, ssem)

        plsc.subcore_barrier()

        rs = pl.ds(sid * ROWS_PER_SUB, ROWS_PER_SUB)

        @pl.when(cid == 0)
        def _():
            pltpu.sync_copy(acc.at[rs], olo.at[rs])

        @pl.when(cid == 1)
        def _():
            pltpu.sync_copy(acc.at[rs], ohi.at[rs])

    return seg_kernel


_segsum1 = _make_segsum1()
_segsum2 = _make_segsum2()

_TC_BLK = 1000


def _sage1(agg_a, agg_b, dega, degb, x, wl, wr, bl):
    def body(aa, ab, da, db, xr, wll, wrr, b, out):
        deg = jnp.maximum(da[...][:, :1] + db[...][:, :1], 1.0)
        h = jnp.dot((aa[...] + ab[...]) / deg, wll[...],
                    preferred_element_type=jnp.float32)
        h = h + jnp.dot(xr[...], wrr[...], preferred_element_type=jnp.float32)
        h = jnp.maximum(h + b[...], 0.0)
        out[0] = h[:, :D_HID // 2]
        out[1] = h[:, D_HID // 2:]

    return pl.pallas_call(
        body,
        grid=(N_NODES // _TC_BLK,),
        in_specs=[
            pl.BlockSpec((_TC_BLK, D_IN), lambda i: (i, 0)),
            pl.BlockSpec((_TC_BLK, D_IN), lambda i: (i, 0)),
            pl.BlockSpec((_TC_BLK, 128), lambda i: (i, 0)),
            pl.BlockSpec((_TC_BLK, 128), lambda i: (i, 0)),
            pl.BlockSpec((_TC_BLK, D_IN), lambda i: (i, 0)),
            pl.BlockSpec((D_IN, D_HID), lambda i: (0, 0)),
            pl.BlockSpec((D_IN, D_HID), lambda i: (0, 0)),
            pl.BlockSpec((1, D_HID), lambda i: (0, 0)),
        ],
        out_specs=pl.BlockSpec((2, _TC_BLK, D_HID // 2), lambda i: (0, i, 0)),
        out_shape=jax.ShapeDtypeStruct((2, N_NODES, D_HID // 2), jnp.float32),
    )(agg_a, agg_b, dega, degb, x, wl, wr, bl)


def _sage2_final(a2lo, a2hi, dega, degb, h1lo, h1hi,
                 wla, wlb, wra, wrb, bl, wf, bf):
    def body(alo, ahi, da, db, hlo, hhi, wa, wb, wr1, wr2, b, wff, bff, out):
        deg = jnp.maximum(da[...][:, :1] + db[...][:, :1], 1.0)
        h = jnp.dot(alo[...] / deg, wa[...], preferred_element_type=jnp.float32)
        h = h + jnp.dot(ahi[...] / deg, wb[...],
                        preferred_element_type=jnp.float32)
        h = h + jnp.dot(hlo[...], wr1[...], preferred_element_type=jnp.float32)
        h = h + jnp.dot(hhi[...], wr2[...], preferred_element_type=jnp.float32)
        h = jnp.maximum(h + b[...], 0.0)
        o = jnp.dot(h, wff[...], preferred_element_type=jnp.float32) + bff[...]
        s = o - jnp.max(o, axis=1, keepdims=True)
        out[...] = s - jnp.log(jnp.sum(jnp.exp(s), axis=1, keepdims=True))

    half = D_HID // 2
    return pl.pallas_call(
        body,
        grid=(N_NODES // _TC_BLK,),
        in_specs=[
            pl.BlockSpec((_TC_BLK, half), lambda i: (i, 0)),
            pl.BlockSpec((_TC_BLK, half), lambda i: (i, 0)),
            pl.BlockSpec((_TC_BLK, 128), lambda i: (i, 0)),
            pl.BlockSpec((_TC_BLK, 128), lambda i: (i, 0)),
            pl.BlockSpec((_TC_BLK, half), lambda i: (i, 0)),
            pl.BlockSpec((_TC_BLK, half), lambda i: (i, 0)),
            pl.BlockSpec((half, D_HID), lambda i: (0, 0)),
            pl.BlockSpec((half, D_HID), lambda i: (0, 0)),
            pl.BlockSpec((half, D_HID), lambda i: (0, 0)),
            pl.BlockSpec((half, D_HID), lambda i: (0, 0)),
            pl.BlockSpec((1, D_HID), lambda i: (0, 0)),
            pl.BlockSpec((D_HID, D_OUT), lambda i: (0, 0)),
            pl.BlockSpec((1, D_OUT), lambda i: (0, 0)),
        ],
        out_specs=pl.BlockSpec((_TC_BLK, D_OUT), lambda i: (i, 0)),
        out_shape=jax.ShapeDtypeStruct((N_NODES, D_OUT), jnp.float32),
    )(a2lo, a2hi, dega, degb, h1lo, h1hi, wla, wlb, wra, wrb, bl, wf, bf)


def kernel(x, edge_index, Wl1, bl1, Wr1, Wl2, bl2, Wr2, Wf, bf):
    src3 = edge_index[0].astype(jnp.int32).reshape(N_BLKS, IDX_BLK, CHUNK)
    dst3 = edge_index[1].astype(jnp.int32).reshape(N_BLKS, IDX_BLK, CHUNK)

    a1a, a1b, dega, degb = _segsum1(src3, dst3, x)
    h13 = _sage1(a1a, a1b, dega, degb, x, Wl1.T, Wr1.T, bl1.reshape(1, -1))
    a2lo, a2hi = _segsum2(src3, dst3, h13[0], h13[1])
    out = _sage2_final(a2lo, a2hi, dega, degb, h13[0], h13[1],
                       Wl2.T[:D_HID // 2], Wl2.T[D_HID // 2:],
                       Wr2.T[:D_HID // 2], Wr2.T[D_HID // 2:],
                       bl2.reshape(1, -1), Wf.T, bf.reshape(1, -1))
    return out
